# Initial kernel scaffold; baseline (speedup 1.0000x reference)
import jax
import jax.numpy as jnp
from jax.experimental import pallas as pl


def kernel(h, edge_index, W, attn_l, attn_r, bias):
    raise NotImplementedError("write your pallas kernel here")


# speedup vs baseline: 64.8373x; 64.8373x over previous
import functools

import jax
import jax.numpy as jnp
from jax import lax
from jax.experimental import pallas as pl
from jax.experimental.pallas import tpu as pltpu
from jax.experimental.pallas import tpu_sc as plsc

N = 10000
E = 320000
D = 128
H = 8
OUT = 16
ROW = 144

NC = 2
NS = 16
NW = NC * NS
EPW = E // NW
B = 80
NCHUNK = EPW // B
NZC = N // B

_LANES = 16


def _lane_bcast(v, lane):
  return jnp.broadcast_to(v[lane], (_LANES,))


def _prep_body(h_ref, w_ref, pl_ref, pr_ref, featel_ref, er_ref):
  feat = jnp.dot(h_ref[...], w_ref[...], preferred_element_type=jnp.float32)
  el16 = jnp.dot(feat, pl_ref[...], preferred_element_type=jnp.float32)
  er16 = jnp.dot(feat, pr_ref[...], preferred_element_type=jnp.float32)
  featel_ref[...] = jnp.concatenate([feat, el16], axis=1)
  er_ref[...] = er16


_PREP_BLK = 1000

_prep = pl.pallas_call(
    _prep_body,
    grid=(N // _PREP_BLK,),
    in_specs=[
        pl.BlockSpec((_PREP_BLK, D), lambda i: (i, 0)),
        pl.BlockSpec((D, D), lambda i: (0, 0)),
        pl.BlockSpec((D, 16), lambda i: (0, 0)),
        pl.BlockSpec((D, 16), lambda i: (0, 0)),
    ],
    out_specs=[
        pl.BlockSpec((_PREP_BLK, ROW), lambda i: (i, 0)),
        pl.BlockSpec((_PREP_BLK, 16), lambda i: (i, 0)),
    ],
    out_shape=[
        jax.ShapeDtypeStruct((N, ROW), jnp.float32),
        jax.ShapeDtypeStruct((N, 16), jnp.float32),
    ],
)


def _sc_body(featel_hbm, er_hbm, src_hbm, dst_hbm, out_hbm,
             acc, sidx, didx, g_buf, r_buf, o_buf, sem_g, sem_r):
  cid = lax.axis_index("c")
  sid = lax.axis_index("s")
  wid = cid * NS + sid

  zv = jnp.zeros((_LANES,), jnp.float32)

  def _zero_row(i, _):
    for c in range(ROW // _LANES):
      o_buf[i, pl.ds(c * _LANES, _LANES)] = zv
    return _

  lax.fori_loop(0, B, _zero_row, None)

  def _zero_chunk(j, _):
    c = sid + j * NS

    @pl.when(c < NZC)
    def _():
      pltpu.sync_copy(o_buf, acc.at[pl.ds(c * B, B)])
    return _

  lax.fori_loop(0, pl.cdiv(NZC, NS), _zero_chunk, None)
  plsc.subcore_barrier()

  def _chunk(c, _):
    off = wid * EPW + c * B
    pltpu.sync_copy(src_hbm.at[pl.ds(off, B)], sidx)
    pltpu.sync_copy(dst_hbm.at[pl.ds(off, B)], didx)
    gcp = pltpu.async_copy(featel_hbm.at[sidx], g_buf, sem_g)
    rcp = pltpu.async_copy(er_hbm.at[didx], r_buf, sem_r)
    gcp.wait()
    rcp.wait()

    def _edge(i, _):
      el = g_buf[i, pl.ds(D, _LANES)]
      er = r_buf[i, pl.ds(0, _LANES)]
      s = el + er
      s = jnp.where(s >= 0.0, s, s * jnp.float32(0.2))
      w = jnp.exp(s)
      o_buf[i, pl.ds(D, _LANES)] = w
      for hh in range(H):
        fh = g_buf[i, pl.ds(hh * OUT, _LANES)]
        o_buf[i, pl.ds(hh * OUT, _LANES)] = fh * _lane_bcast(w, hh)
      return _

    lax.fori_loop(0, B, _edge, None)
    pltpu.sync_copy(o_buf, acc.at[didx], add=True)
    return _

  lax.fori_loop(0, NCHUNK, _chunk, None)
  plsc.subcore_barrier()

  def _out_chunk(j, _):
    c = sid + j * NS

    @pl.when(c < NZC)
    def _():
      pltpu.sync_copy(acc.at[pl.ds(c * B, B)],
                      out_hbm.at[cid, pl.ds(c * B, B)])
    return _

  lax.fori_loop(0, pl.cdiv(NZC, NS), _out_chunk, None)


@functools.cache
def _make_sc_edge():
  return pl.kernel(
      _sc_body,
      out_type=jax.ShapeDtypeStruct((NC, N, ROW), jnp.float32),
      mesh=plsc.VectorSubcoreMesh(
          core_axis_name="c", subcore_axis_name="s",
          num_cores=NC, num_subcores=NS),
      scratch_types=[
          pltpu.VMEM_SHARED((N, ROW), jnp.float32),
          pltpu.VMEM((B,), jnp.int32),
          pltpu.VMEM((B,), jnp.int32),
          pltpu.VMEM((B, ROW), jnp.float32),
          pltpu.VMEM((B, 16), jnp.float32),
          pltpu.VMEM((B, ROW), jnp.float32),
          pltpu.SemaphoreType.DMA,
          pltpu.SemaphoreType.DMA,
      ],
      compiler_params=pltpu.CompilerParams(use_tc_tiling_on_sc=False),
  )


def _fin_body(p0_ref, p1_ref, b_ref, out_ref):
  a0 = p0_ref[...]
  a1 = p1_ref[...]
  s = a0[:, :D] + a1[:, :D]
  d8 = a0[:, D:D + H] + a1[:, D:D + H]
  hh = lax.broadcasted_iota(jnp.int32, (H, D), 0)
  jj = lax.broadcasted_iota(jnp.int32, (H, D), 1)
  expand = (jj // OUT == hh).astype(jnp.float32)
  drep = jnp.dot(d8, expand, preferred_element_type=jnp.float32)
  x = s / jnp.maximum(drep, jnp.float32(1e-38)) + b_ref[...]
  out_ref[...] = jnp.where(x > 0.0, x, jnp.exp(x) - 1.0)


_fin = pl.pallas_call(
    _fin_body,
    grid=(N // _PREP_BLK,),
    in_specs=[
        pl.BlockSpec((_PREP_BLK, ROW), lambda i: (i, 0)),
        pl.BlockSpec((_PREP_BLK, ROW), lambda i: (i, 0)),
        pl.BlockSpec((1, D), lambda i: (0, 0)),
    ],
    out_specs=pl.BlockSpec((_PREP_BLK, D), lambda i: (i, 0)),
    out_shape=jax.ShapeDtypeStruct((N, D), jnp.float32),
)


def kernel(h, edge_index, W, attn_l, attn_r, bias):
  src = edge_index[0].astype(jnp.int32)
  dst = edge_index[1].astype(jnp.int32)

  rows = jnp.arange(D)
  cols = rows // OUT
  p_l = jnp.zeros((D, 16), jnp.float32).at[rows, cols].set(attn_l.reshape(-1))
  p_r = jnp.zeros((D, 16), jnp.float32).at[rows, cols].set(attn_r.reshape(-1))

  featel, er16 = _prep(h, W, p_l, p_r)
  partials = _make_sc_edge()(featel, er16, src, dst)
  out = _fin(partials[0], partials[1], bias.reshape(1, D))
  return out
